# baseline (speedup 1.0000x reference)
import functools
import math

import jax
import jax.numpy as jnp
from jax import lax
from jax.experimental import pallas as pl
from jax.experimental.pallas import tpu as pltpu
from jax.experimental.pallas import tpu_sc as plsc

N = 10000
E = 320000
D = 128

NC = 2
NS = 16
NW = NC * NS
EPW = E // NW
WIN = 96
NWINF = EPW // WIN
TAIL = EPW - NWINF * WIN
NP = 10240
RPT = NP // NS

ALPHA = 0.5
BETA1 = math.log(1.0 / 1.0 + 1.0)
BETA2 = math.log(1.0 / 2.0 + 1.0)

_mesh = plsc.VectorSubcoreMesh(core_axis_name="c", subcore_axis_name="s")
_sc_params = pltpu.CompilerParams(use_tc_tiling_on_sc=False)


@functools.partial(
    pl.kernel,
    out_type=jax.ShapeDtypeStruct((NC, NP), jnp.float32),
    mesh=_mesh,
    scratch_types=[
        pltpu.VMEM((EPW,), jnp.int32),
        pltpu.VMEM((1, WIN), jnp.int32),
        pltpu.VMEM((WIN,), jnp.float32),
        pltpu.VMEM((RPT,), jnp.float32),
        pltpu.VMEM_SHARED((NP,), jnp.float32),
    ],
    compiler_params=_sc_params,
)
def _deg_sc(dst_hbm, out_hbm, idx_v, idx_w, ones_v, zbuf_v, acc_sh):
    c = lax.axis_index("c")
    s = lax.axis_index("s")
    w = c * NS + s
    for i in range(WIN // 16):
        ones_v[pl.ds(i * 16, 16)] = jnp.ones((16,), jnp.float32)

    def zstore(q, carry):
        zbuf_v[pl.ds(q * 16, 16)] = jnp.zeros((16,), jnp.float32)
        return carry

    lax.fori_loop(0, RPT // 16, zstore, 0)
    pltpu.sync_copy(zbuf_v, acc_sh.at[pl.ds(s * RPT, RPT)])
    pltpu.sync_copy(dst_hbm.at[pl.ds(w * EPW, EPW)], idx_v)
    plsc.subcore_barrier()

    def body(j, carry):
        for k in range(WIN // 16):
            idx_w[0, pl.ds(k * 16, 16)] = idx_v[pl.ds(j * WIN + k * 16, 16)]
        pltpu.sync_copy(ones_v, acc_sh.at[idx_w.at[0]], add=True)
        return carry

    lax.fori_loop(0, NWINF, body, 0)
    tail_idx = idx_v[pl.ds(NWINF * WIN, TAIL)]
    pltpu.sync_copy(ones_v.at[pl.ds(0, TAIL)],
                    acc_sh.at[tail_idx], add=True)
    plsc.subcore_barrier()
    pltpu.sync_copy(acc_sh.at[pl.ds(s * RPT, RPT)], out_hbm.at[c, pl.ds(s * RPT, RPT)])


@functools.partial(
    pl.kernel,
    out_type=jax.ShapeDtypeStruct((NC, NP, D), jnp.float32),
    mesh=_mesh,
    scratch_types=[
        pltpu.VMEM((EPW,), jnp.int32),
        pltpu.VMEM((EPW,), jnp.int32),
        pltpu.VMEM((2, WIN), jnp.int32),
        pltpu.VMEM((2, WIN, D), jnp.float32),
        pltpu.VMEM_SHARED((NP, D), jnp.float32),
        pltpu.SemaphoreType.DMA,
        pltpu.SemaphoreType.DMA,
    ],
    compiler_params=_sc_params,
)
def _agg_sc(h_hbm, src_hbm, dst_hbm, zeros_hbm, out_hbm,
            src_v, dst_v, dst_w, rows_v, acc_sh, gsem0, gsem1):
    c = lax.axis_index("c")
    s = lax.axis_index("s")
    w = c * NS + s
    pltpu.sync_copy(zeros_hbm.at[pl.ds(s * RPT, RPT)], acc_sh.at[pl.ds(s * RPT, RPT)])
    pltpu.sync_copy(src_hbm.at[pl.ds(w * EPW, EPW)], src_v)
    pltpu.sync_copy(dst_hbm.at[pl.ds(w * EPW, EPW)], dst_v)
    plsc.subcore_barrier()

    def _start(j, b, sem):
        pltpu.async_copy(h_hbm.at[src_v.at[pl.ds(j * WIN, WIN)]],
                         rows_v.at[b], sem)

    def _drain(j, b, sem):
        pltpu.make_async_copy(h_hbm.at[src_v.at[pl.ds(j * WIN, WIN)]],
                              rows_v.at[b], sem).wait()
        for k in range(WIN // 16):
            dst_w[b, pl.ds(k * 16, 16)] = dst_v[pl.ds(j * WIN + k * 16, 16)]
        pltpu.sync_copy(rows_v.at[b], acc_sh.at[dst_w.at[b]], add=True)

    _start(0, 0, gsem0)
    _start(1, 1, gsem1)

    def body(i, carry):
        j = 2 * i
        _drain(j, 0, gsem0)
        _start(j + 2, 0, gsem0)
        _drain(j + 1, 1, gsem1)
        _start(j + 3, 1, gsem1)
        return carry

    lax.fori_loop(0, NWINF // 2 - 1, body, 0)
    _drain(NWINF - 2, 0, gsem0)
    _drain(NWINF - 1, 1, gsem1)
    t0 = NWINF * WIN
    tail_src = src_v[pl.ds(t0, TAIL)]
    pltpu.sync_copy(h_hbm.at[tail_src], rows_v.at[1, pl.ds(0, TAIL)])
    tail_dst = dst_v[pl.ds(t0, TAIL)]
    pltpu.sync_copy(rows_v.at[1, pl.ds(0, TAIL)],
                    acc_sh.at[tail_dst], add=True)

    plsc.subcore_barrier()
    pltpu.sync_copy(acc_sh.at[pl.ds(s * RPT, RPT)], out_hbm.at[c, pl.ds(s * RPT, RPT)])


def _leaky(v):
    return jnp.where(v >= 0, v, 0.01 * v)


def _norm_from_deg(deg_ref):
    deg = deg_ref[0, :N] + deg_ref[1, :N]
    return jnp.where(deg > 0, lax.rsqrt(jnp.maximum(deg, 1.0)), 0.0)


def _split_body(edge_ref, src_ref, dst_ref):
    src_ref[...] = edge_ref[0, :]
    dst_ref[...] = edge_ref[1, :]


_split = pl.pallas_call(
    _split_body,
    out_shape=[
        jax.ShapeDtypeStruct((E,), jnp.int32),
        jax.ShapeDtypeStruct((E,), jnp.int32),
    ],
)


def _tc1_body(deg_ref, x_ref, h1n_ref):
    norm = _norm_from_deg(deg_ref)
    h1n_ref[...] = x_ref[...] * norm[:, None]


_tc1 = pl.pallas_call(
    _tc1_body,
    out_shape=jax.ShapeDtypeStruct((N, D), jnp.float32),
)


def _tc2_body(aggp_ref, x_ref, deg_ref, W1_ref, b1_ref, o1_ref, h2n_ref):
    norm = _norm_from_deg(deg_ref)
    agg = (aggp_ref[0, :N] + aggp_ref[1, :N]) * norm[:, None]
    t = (1.0 - ALPHA) * agg + ALPHA * x_ref[...]
    z = (1.0 - BETA1) * t + BETA1 * jnp.dot(
        t, W1_ref[...], preferred_element_type=jnp.float32) + b1_ref[...][None, :]
    o1 = _leaky(z)
    o1_ref[...] = o1
    h2n_ref[...] = o1 * norm[:, None]


_tc2 = pl.pallas_call(
    _tc2_body,
    out_shape=[
        jax.ShapeDtypeStruct((N, D), jnp.float32),
        jax.ShapeDtypeStruct((N, D), jnp.float32),
    ],
)


def _tc3_body(aggp_ref, o1_ref, deg_ref, W2_ref, b2_ref, g_ref, bb_ref,
              f1w_ref, f1b_ref, f2w_ref, f2b_ref, out_ref):
    norm = _norm_from_deg(deg_ref)
    agg = (aggp_ref[0, :N] + aggp_ref[1, :N]) * norm[:, None]
    t = (1.0 - ALPHA) * agg + ALPHA * o1_ref[...]
    h = (1.0 - BETA2) * t + BETA2 * jnp.dot(
        t, W2_ref[...], preferred_element_type=jnp.float32) + b2_ref[...][None, :]
    mean = jnp.mean(h, axis=0)
    var = jnp.mean((h - mean[None, :]) ** 2, axis=0)
    hb = (h - mean[None, :]) / jnp.sqrt(var + 1e-5)[None, :] * g_ref[...][None, :] \
        + bb_ref[...][None, :]
    hb = _leaky(hb)
    pooled = jnp.sum(hb, axis=0, keepdims=True)
    u = _leaky(jnp.dot(pooled, f1w_ref[...], preferred_element_type=jnp.float32)
               + f1b_ref[...][None, :])
    out_ref[...] = jnp.dot(u, f2w_ref[...], preferred_element_type=jnp.float32) \
        + f2b_ref[...][None, :]


_tc3 = pl.pallas_call(
    _tc3_body,
    out_shape=jax.ShapeDtypeStruct((1, 2), jnp.float32),
)


def kernel(x, edge_index, edge_weights, W1, b1, W2, b2, bn_gamma, bn_beta,
           fc1_W, fc1_b, fc2_W, fc2_b):
    del edge_weights
    src, dst = _split(edge_index)

    zeros_nd = jnp.zeros((NP, D), jnp.float32)
    deg_parts = _deg_sc(dst)
    h1n = _tc1(deg_parts, x)
    agg1 = _agg_sc(h1n, src, dst, zeros_nd)
    o1, h2n = _tc2(agg1, x, deg_parts, W1, b1)
    agg2 = _agg_sc(h2n, src, dst, zeros_nd)
    out = _tc3(agg2, o1, deg_parts, W2, b2, bn_gamma, bn_beta,
               fc1_W, fc1_b, fc2_W, fc2_b)
    return out


# speedup vs baseline: 1.0207x; 1.0207x over previous
import functools
import math

import jax
import jax.numpy as jnp
from jax import lax
from jax.experimental import pallas as pl
from jax.experimental.pallas import tpu as pltpu
from jax.experimental.pallas import tpu_sc as plsc

N = 10000
E = 320000
D = 128

NC = 2
NS = 16
NW = NC * NS
EPW = E // NW
WIN = 96
NWINF = EPW // WIN
TAIL = EPW - NWINF * WIN
NP = 10240
RPT = NP // NS

DWIN = 2000
NDWIN = EPW // DWIN

ALPHA = 0.5
BETA1 = math.log(1.0 / 1.0 + 1.0)
BETA2 = math.log(1.0 / 2.0 + 1.0)

_mesh = plsc.VectorSubcoreMesh(core_axis_name="c", subcore_axis_name="s")
_sc_params = pltpu.CompilerParams(use_tc_tiling_on_sc=False)


@functools.partial(
    pl.kernel,
    out_type=jax.ShapeDtypeStruct((NC, NP), jnp.float32),
    mesh=_mesh,
    scratch_types=[
        pltpu.VMEM((EPW,), jnp.int32),
        pltpu.VMEM((1, DWIN), jnp.int32),
        pltpu.VMEM((DWIN,), jnp.float32),
        pltpu.VMEM((RPT,), jnp.float32),
        pltpu.VMEM_SHARED((NP,), jnp.float32),
    ],
    compiler_params=_sc_params,
)
def _deg_sc(dst_hbm, out_hbm, idx_v, idx_w, ones_v, zbuf_v, acc_sh):
    c = lax.axis_index("c")
    s = lax.axis_index("s")
    w = c * NS + s

    def ofill(q, carry):
        ones_v[pl.ds(q * 16, 16)] = jnp.ones((16,), jnp.float32)
        return carry

    lax.fori_loop(0, DWIN // 16, ofill, 0)

    def zstore(q, carry):
        zbuf_v[pl.ds(q * 16, 16)] = jnp.zeros((16,), jnp.float32)
        return carry

    lax.fori_loop(0, RPT // 16, zstore, 0)
    pltpu.sync_copy(zbuf_v, acc_sh.at[pl.ds(s * RPT, RPT)])
    pltpu.sync_copy(dst_hbm.at[pl.ds(w * EPW, EPW)], idx_v)
    plsc.subcore_barrier()

    def body(j, carry):
        def mirror(k, carry2):
            idx_w[0, pl.ds(k * 16, 16)] = idx_v[pl.ds(j * DWIN + k * 16, 16)]
            return carry2

        lax.fori_loop(0, DWIN // 16, mirror, 0)
        pltpu.sync_copy(ones_v, acc_sh.at[idx_w.at[0]], add=True)
        return carry

    lax.fori_loop(0, NDWIN, body, 0)
    plsc.subcore_barrier()
    pltpu.sync_copy(acc_sh.at[pl.ds(s * RPT, RPT)], out_hbm.at[c, pl.ds(s * RPT, RPT)])


@functools.partial(
    pl.kernel,
    out_type=jax.ShapeDtypeStruct((NC, NP, D), jnp.float32),
    mesh=_mesh,
    scratch_types=[
        pltpu.VMEM((EPW,), jnp.int32),
        pltpu.VMEM((EPW,), jnp.int32),
        pltpu.VMEM((2, WIN), jnp.int32),
        pltpu.VMEM((2, WIN, D), jnp.float32),
        pltpu.VMEM_SHARED((NP, D), jnp.float32),
        pltpu.SemaphoreType.DMA,
        pltpu.SemaphoreType.DMA,
    ],
    compiler_params=_sc_params,
)
def _agg_sc(h_hbm, src_hbm, dst_hbm, zeros_hbm, out_hbm,
            src_v, dst_v, dst_w, rows_v, acc_sh, gsem0, gsem1):
    c = lax.axis_index("c")
    s = lax.axis_index("s")
    w = c * NS + s
    pltpu.sync_copy(zeros_hbm.at[pl.ds(s * RPT, RPT)], acc_sh.at[pl.ds(s * RPT, RPT)])
    pltpu.sync_copy(src_hbm.at[pl.ds(w * EPW, EPW)], src_v)
    pltpu.sync_copy(dst_hbm.at[pl.ds(w * EPW, EPW)], dst_v)
    plsc.subcore_barrier()

    def _start(j, b, sem):
        pltpu.async_copy(h_hbm.at[src_v.at[pl.ds(j * WIN, WIN)]],
                         rows_v.at[b], sem)

    def _drain(j, b, sem):
        pltpu.make_async_copy(h_hbm.at[src_v.at[pl.ds(j * WIN, WIN)]],
                              rows_v.at[b], sem).wait()
        for k in range(WIN // 16):
            dst_w[b, pl.ds(k * 16, 16)] = dst_v[pl.ds(j * WIN + k * 16, 16)]
        pltpu.sync_copy(rows_v.at[b], acc_sh.at[dst_w.at[b]], add=True)

    _start(0, 0, gsem0)
    _start(1, 1, gsem1)

    def body(i, carry):
        j = 2 * i
        _drain(j, 0, gsem0)
        _start(j + 2, 0, gsem0)
        _drain(j + 1, 1, gsem1)
        _start(j + 3, 1, gsem1)
        return carry

    lax.fori_loop(0, NWINF // 2 - 1, body, 0)
    _drain(NWINF - 2, 0, gsem0)
    _drain(NWINF - 1, 1, gsem1)
    t0 = NWINF * WIN
    tail_src = src_v[pl.ds(t0, TAIL)]
    pltpu.sync_copy(h_hbm.at[tail_src], rows_v.at[1, pl.ds(0, TAIL)])
    tail_dst = dst_v[pl.ds(t0, TAIL)]
    pltpu.sync_copy(rows_v.at[1, pl.ds(0, TAIL)],
                    acc_sh.at[tail_dst], add=True)

    plsc.subcore_barrier()
    pltpu.sync_copy(acc_sh.at[pl.ds(s * RPT, RPT)], out_hbm.at[c, pl.ds(s * RPT, RPT)])


def _leaky(v):
    return jnp.where(v >= 0, v, 0.01 * v)


def _norm_from_deg(deg_ref):
    deg = deg_ref[0, :N] + deg_ref[1, :N]
    return jnp.where(deg > 0, lax.rsqrt(jnp.maximum(deg, 1.0)), 0.0)


def _split_body(edge_ref, src_ref, dst_ref):
    src_ref[...] = edge_ref[0, :]
    dst_ref[...] = edge_ref[1, :]


_split = pl.pallas_call(
    _split_body,
    out_shape=[
        jax.ShapeDtypeStruct((E,), jnp.int32),
        jax.ShapeDtypeStruct((E,), jnp.int32),
    ],
)


def _tc1_body(deg_ref, x_ref, h1n_ref):
    norm = _norm_from_deg(deg_ref)
    h1n_ref[...] = x_ref[...] * norm[:, None]


_tc1 = pl.pallas_call(
    _tc1_body,
    out_shape=jax.ShapeDtypeStruct((N, D), jnp.float32),
)


def _tc2_body(aggp_ref, x_ref, deg_ref, W1_ref, b1_ref, o1_ref, h2n_ref):
    norm = _norm_from_deg(deg_ref)
    agg = (aggp_ref[0, :N] + aggp_ref[1, :N]) * norm[:, None]
    t = (1.0 - ALPHA) * agg + ALPHA * x_ref[...]
    z = (1.0 - BETA1) * t + BETA1 * jnp.dot(
        t, W1_ref[...], preferred_element_type=jnp.float32) + b1_ref[...][None, :]
    o1 = _leaky(z)
    o1_ref[...] = o1
    h2n_ref[...] = o1 * norm[:, None]


_tc2 = pl.pallas_call(
    _tc2_body,
    out_shape=[
        jax.ShapeDtypeStruct((N, D), jnp.float32),
        jax.ShapeDtypeStruct((N, D), jnp.float32),
    ],
)


def _tc3_body(aggp_ref, o1_ref, deg_ref, W2_ref, b2_ref, g_ref, bb_ref,
              f1w_ref, f1b_ref, f2w_ref, f2b_ref, out_ref):
    norm = _norm_from_deg(deg_ref)
    agg = (aggp_ref[0, :N] + aggp_ref[1, :N]) * norm[:, None]
    t = (1.0 - ALPHA) * agg + ALPHA * o1_ref[...]
    h = (1.0 - BETA2) * t + BETA2 * jnp.dot(
        t, W2_ref[...], preferred_element_type=jnp.float32) + b2_ref[...][None, :]
    mean = jnp.mean(h, axis=0)
    var = jnp.mean((h - mean[None, :]) ** 2, axis=0)
    hb = (h - mean[None, :]) / jnp.sqrt(var + 1e-5)[None, :] * g_ref[...][None, :] \
        + bb_ref[...][None, :]
    hb = _leaky(hb)
    pooled = jnp.sum(hb, axis=0, keepdims=True)
    u = _leaky(jnp.dot(pooled, f1w_ref[...], preferred_element_type=jnp.float32)
               + f1b_ref[...][None, :])
    out_ref[...] = jnp.dot(u, f2w_ref[...], preferred_element_type=jnp.float32) \
        + f2b_ref[...][None, :]


_tc3 = pl.pallas_call(
    _tc3_body,
    out_shape=jax.ShapeDtypeStruct((1, 2), jnp.float32),
)


def kernel(x, edge_index, edge_weights, W1, b1, W2, b2, bn_gamma, bn_beta,
           fc1_W, fc1_b, fc2_W, fc2_b):
    del edge_weights
    src, dst = _split(edge_index)

    zeros_nd = jnp.zeros((NP, D), jnp.float32)
    deg_parts = _deg_sc(dst)
    h1n = _tc1(deg_parts, x)
    agg1 = _agg_sc(h1n, src, dst, zeros_nd)
    o1, h2n = _tc2(agg1, x, deg_parts, W1, b1)
    agg2 = _agg_sc(h2n, src, dst, zeros_nd)
    out = _tc3(agg2, o1, deg_parts, W2, b2, bn_gamma, bn_beta,
               fc1_W, fc1_b, fc2_W, fc2_b)
    return out


# speedup vs baseline: 1.0672x; 1.0455x over previous
import functools
import math

import jax
import jax.numpy as jnp
from jax import lax
from jax.experimental import pallas as pl
from jax.experimental.pallas import tpu as pltpu
from jax.experimental.pallas import tpu_sc as plsc

N = 10000
E = 320000
D = 128

NC = 2
NS = 16
NW = NC * NS
EPW = E // NW
WIN = 128
NWINF = EPW // WIN
TAIL = EPW - NWINF * WIN
NP = 10240
RPT = NP // NS

DWIN = 2000
NDWIN = EPW // DWIN

ALPHA = 0.5
BETA1 = math.log(1.0 / 1.0 + 1.0)
BETA2 = math.log(1.0 / 2.0 + 1.0)

_mesh = plsc.VectorSubcoreMesh(core_axis_name="c", subcore_axis_name="s")
_sc_params = pltpu.CompilerParams(use_tc_tiling_on_sc=False)


@functools.partial(
    pl.kernel,
    out_type=jax.ShapeDtypeStruct((NC, NP), jnp.float32),
    mesh=_mesh,
    scratch_types=[
        pltpu.VMEM((EPW,), jnp.int32),
        pltpu.VMEM((1, DWIN), jnp.int32),
        pltpu.VMEM((DWIN,), jnp.float32),
        pltpu.VMEM((RPT,), jnp.float32),
        pltpu.VMEM_SHARED((NP,), jnp.float32),
    ],
    compiler_params=_sc_params,
)
def _deg_sc(dst_hbm, out_hbm, idx_v, idx_w, ones_v, zbuf_v, acc_sh):
    c = lax.axis_index("c")
    s = lax.axis_index("s")
    w = c * NS + s

    def ofill(q, carry):
        ones_v[pl.ds(q * 16, 16)] = jnp.ones((16,), jnp.float32)
        return carry

    lax.fori_loop(0, DWIN // 16, ofill, 0)

    def zstore(q, carry):
        zbuf_v[pl.ds(q * 16, 16)] = jnp.zeros((16,), jnp.float32)
        return carry

    lax.fori_loop(0, RPT // 16, zstore, 0)
    pltpu.sync_copy(zbuf_v, acc_sh.at[pl.ds(s * RPT, RPT)])
    pltpu.sync_copy(dst_hbm.at[pl.ds(w * EPW, EPW)], idx_v)
    plsc.subcore_barrier()

    def body(j, carry):
        def mirror(k, carry2):
            idx_w[0, pl.ds(k * 16, 16)] = idx_v[pl.ds(j * DWIN + k * 16, 16)]
            return carry2

        lax.fori_loop(0, DWIN // 16, mirror, 0)
        pltpu.sync_copy(ones_v, acc_sh.at[idx_w.at[0]], add=True)
        return carry

    lax.fori_loop(0, NDWIN, body, 0)
    plsc.subcore_barrier()
    pltpu.sync_copy(acc_sh.at[pl.ds(s * RPT, RPT)], out_hbm.at[c, pl.ds(s * RPT, RPT)])


@functools.partial(
    pl.kernel,
    out_type=jax.ShapeDtypeStruct((NC, NP, D), jnp.float32),
    mesh=_mesh,
    scratch_types=[
        pltpu.VMEM((EPW,), jnp.int32),
        pltpu.VMEM((2, WIN), jnp.int32),
        pltpu.VMEM((2, WIN, D), jnp.float32),
        pltpu.VMEM_SHARED((NP, D), jnp.float32),
        pltpu.SemaphoreType.DMA,
        pltpu.SemaphoreType.DMA,
        pltpu.SemaphoreType.DMA,
        pltpu.SemaphoreType.DMA,
    ],
    compiler_params=_sc_params,
)
def _agg_sc(h_hbm, src_hbm, dst_hbm, zeros_hbm, out_hbm,
            src_v, dst_w, rows_v, acc_sh, gsem0, gsem1, isem0, isem1):
    c = lax.axis_index("c")
    s = lax.axis_index("s")
    w = c * NS + s
    pltpu.sync_copy(zeros_hbm.at[pl.ds(s * RPT, RPT)], acc_sh.at[pl.ds(s * RPT, RPT)])
    pltpu.sync_copy(src_hbm.at[pl.ds(w * EPW, EPW)], src_v)
    plsc.subcore_barrier()

    def _start(j, b, sem, isem):
        pltpu.async_copy(h_hbm.at[src_v.at[pl.ds(j * WIN, WIN)]],
                         rows_v.at[b], sem)
        pltpu.async_copy(dst_hbm.at[pl.ds(w * EPW + j * WIN, WIN)],
                         dst_w.at[b], isem)

    def _drain(j, b, sem, isem):
        pltpu.make_async_copy(h_hbm.at[src_v.at[pl.ds(j * WIN, WIN)]],
                              rows_v.at[b], sem).wait()
        pltpu.make_async_copy(dst_hbm.at[pl.ds(w * EPW + j * WIN, WIN)],
                              dst_w.at[b], isem).wait()
        pltpu.sync_copy(rows_v.at[b], acc_sh.at[dst_w.at[b]], add=True)

    _start(0, 0, gsem0, isem0)
    _start(1, 1, gsem1, isem1)

    def body(i, carry):
        j = 2 * i
        _drain(j, 0, gsem0, isem0)
        _start(j + 2, 0, gsem0, isem0)
        _drain(j + 1, 1, gsem1, isem1)
        _start(j + 3, 1, gsem1, isem1)
        return carry

    lax.fori_loop(0, NWINF // 2 - 1, body, 0)
    _drain(NWINF - 2, 0, gsem0, isem0)
    _drain(NWINF - 1, 1, gsem1, isem1)
    t0 = NWINF * WIN
    tail_src = src_v[pl.ds(t0, TAIL)]
    pltpu.sync_copy(h_hbm.at[tail_src], rows_v.at[1, pl.ds(0, TAIL)])
    pltpu.sync_copy(dst_hbm.at[pl.ds(w * EPW + t0, TAIL)],
                    dst_w.at[0, pl.ds(0, TAIL)])
    tail_dst = dst_w[0, pl.ds(0, TAIL)]
    pltpu.sync_copy(rows_v.at[1, pl.ds(0, TAIL)],
                    acc_sh.at[tail_dst], add=True)

    plsc.subcore_barrier()
    pltpu.sync_copy(acc_sh.at[pl.ds(s * RPT, RPT)], out_hbm.at[c, pl.ds(s * RPT, RPT)])


def _leaky(v):
    return jnp.where(v >= 0, v, 0.01 * v)


def _norm_from_deg(deg_ref):
    deg = deg_ref[0, :N] + deg_ref[1, :N]
    return jnp.where(deg > 0, lax.rsqrt(jnp.maximum(deg, 1.0)), 0.0)


def _split_body(edge_ref, src_ref, dst_ref):
    src_ref[...] = edge_ref[0, :]
    dst_ref[...] = edge_ref[1, :]


_split = pl.pallas_call(
    _split_body,
    out_shape=[
        jax.ShapeDtypeStruct((E,), jnp.int32),
        jax.ShapeDtypeStruct((E,), jnp.int32),
    ],
)


def _tc1_body(deg_ref, x_ref, h1n_ref):
    norm = _norm_from_deg(deg_ref)
    h1n_ref[...] = x_ref[...] * norm[:, None]


_tc1 = pl.pallas_call(
    _tc1_body,
    out_shape=jax.ShapeDtypeStruct((N, D), jnp.float32),
)


def _tc2_body(aggp_ref, x_ref, deg_ref, W1_ref, b1_ref, o1_ref, h2n_ref):
    norm = _norm_from_deg(deg_ref)
    agg = (aggp_ref[0, :N] + aggp_ref[1, :N]) * norm[:, None]
    t = (1.0 - ALPHA) * agg + ALPHA * x_ref[...]
    z = (1.0 - BETA1) * t + BETA1 * jnp.dot(
        t, W1_ref[...], preferred_element_type=jnp.float32) + b1_ref[...][None, :]
    o1 = _leaky(z)
    o1_ref[...] = o1
    h2n_ref[...] = o1 * norm[:, None]


_tc2 = pl.pallas_call(
    _tc2_body,
    out_shape=[
        jax.ShapeDtypeStruct((N, D), jnp.float32),
        jax.ShapeDtypeStruct((N, D), jnp.float32),
    ],
)


def _tc3_body(aggp_ref, o1_ref, deg_ref, W2_ref, b2_ref, g_ref, bb_ref,
              f1w_ref, f1b_ref, f2w_ref, f2b_ref, out_ref):
    norm = _norm_from_deg(deg_ref)
    agg = (aggp_ref[0, :N] + aggp_ref[1, :N]) * norm[:, None]
    t = (1.0 - ALPHA) * agg + ALPHA * o1_ref[...]
    h = (1.0 - BETA2) * t + BETA2 * jnp.dot(
        t, W2_ref[...], preferred_element_type=jnp.float32) + b2_ref[...][None, :]
    mean = jnp.mean(h, axis=0)
    var = jnp.mean((h - mean[None, :]) ** 2, axis=0)
    hb = (h - mean[None, :]) / jnp.sqrt(var + 1e-5)[None, :] * g_ref[...][None, :] \
        + bb_ref[...][None, :]
    hb = _leaky(hb)
    pooled = jnp.sum(hb, axis=0, keepdims=True)
    u = _leaky(jnp.dot(pooled, f1w_ref[...], preferred_element_type=jnp.float32)
               + f1b_ref[...][None, :])
    out_ref[...] = jnp.dot(u, f2w_ref[...], preferred_element_type=jnp.float32) \
        + f2b_ref[...][None, :]


_tc3 = pl.pallas_call(
    _tc3_body,
    out_shape=jax.ShapeDtypeStruct((1, 2), jnp.float32),
)


def kernel(x, edge_index, edge_weights, W1, b1, W2, b2, bn_gamma, bn_beta,
           fc1_W, fc1_b, fc2_W, fc2_b):
    del edge_weights
    src, dst = _split(edge_index)

    zeros_nd = jnp.zeros((NP, D), jnp.float32)
    deg_parts = _deg_sc(dst)
    h1n = _tc1(deg_parts, x)
    agg1 = _agg_sc(h1n, src, dst, zeros_nd)
    o1, h2n = _tc2(agg1, x, deg_parts, W1, b1)
    agg2 = _agg_sc(h2n, src, dst, zeros_nd)
    out = _tc3(agg2, o1, deg_parts, W2, b2, bn_gamma, bn_beta,
               fc1_W, fc1_b, fc2_W, fc2_b)
    return out
